# baseline (speedup 1.0000x reference)
import functools

import jax
import jax.numpy as jnp
from jax import lax
from jax.experimental import pallas as pl
from jax.experimental.pallas import tpu as pltpu
from jax.experimental.pallas import tpu_sc as plsc

NU = 25000
NI = 25000
NN = NU + NI
D = 64
K = 8
PAD = 100000
NL = 3
E = 800000
NC = 2
NS = 16

_MESH = plsc.VectorSubcoreMesh(core_axis_name="c", subcore_axis_name="s")

IG = 56
NG = 14
IW = IG * NG


def _kg_body(user_hbm, item_hbm, ent_hbm, ief_hbm, e0_hbm,
             idx_v, rows_v, item_v, out_v, ubuf_v):
    c = lax.axis_index("c")
    s = lax.axis_index("s")
    w = s * NC + c

    ub = pl.multiple_of(jnp.minimum(w * IW, NU - IW), 8)
    for t in range(2):
        pltpu.sync_copy(user_hbm.at[pl.ds(ub + t * 392, 392)], ubuf_v)
        pltpu.sync_copy(ubuf_v, e0_hbm.at[pl.ds(ub + t * 392, 392)])

    base = pl.multiple_of(jnp.minimum(w * IW, NI - IW), 8)

    def group(g, _):
        ib = pl.multiple_of(base + g * IG, 8)
        pltpu.sync_copy(ief_hbm.at[pl.ds(ib * K, IG * K)],
                        idx_v.at[pl.ds(0, IG * K)])
        pltpu.sync_copy(ent_hbm.at[idx_v.at[pl.ds(0, IG * K)]], rows_v)
        pltpu.sync_copy(item_hbm.at[pl.ds(ib, IG)], item_v)

        def item(i, _):
            acc = [jnp.zeros((16,), jnp.float32) for _ in range(4)]
            den = jnp.float32(0.0)
            nv = idx_v[pl.ds(i * K, 16)]
            mv = jnp.where(nv != PAD, jnp.float32(1.0), jnp.float32(0.0))
            for k in range(K):
                m = mv[k]
                den = den + m
                for q in range(4):
                    acc[q] = acc[q] + m * rows_v[i * K + k, pl.ds(q * 16, 16)]
            denv = jnp.full((16,), den, jnp.float32)
            invv = jnp.where(denv > 0.0, jnp.float32(1.0) / denv,
                             jnp.float32(0.0))
            for q in range(4):
                out_v[i, pl.ds(q * 16, 16)] = (
                    acc[q] * invv + item_v[i, pl.ds(q * 16, 16)])
            return 0

        lax.fori_loop(0, IG, item, 0)
        pltpu.sync_copy(out_v, e0_hbm.at[pl.ds(NU + ib, IG)])
        return 0

    lax.fori_loop(0, NG, group, 0)


_SC_PARAMS = pltpu.CompilerParams(use_tc_tiling_on_sc=False,
                                  needs_layout_passes=False)

_kg_call = pl.kernel(
    _kg_body,
    out_type=jax.ShapeDtypeStruct((NN, D), jnp.float32),
    mesh=_MESH,
    compiler_params=_SC_PARAMS,
    scratch_types=[
        pltpu.VMEM((IG * K + 8,), jnp.int32),
        pltpu.VMEM((IG * K, D), jnp.float32),
        pltpu.VMEM((IG, D), jnp.float32),
        pltpu.VMEM((IG, D), jnp.float32),
        pltpu.VMEM((392, D), jnp.float32),
    ],
)


CAP = 25600
BLK = 1000
STG = 1016


def _prep_body(ei_hbm, ev_hbm, psrc_hbm, padj_hbm, pval_hbm, pcnt_hbm,
               bsrc, bdst, bval, ss0, sa0, sv0, ss1, sa1, sv1, cbuf,
               cnt0v, cnt1v, off0v, off1v):
    c = lax.axis_index("c")
    s = lax.axis_index("s")
    w = s * NC + c
    ssb = (ss0, ss1)
    sab = (sa0, sa1)
    svb = (sv0, sv1)
    cntv = (cnt0v, cnt1v)
    offv = (off0v, off1v)
    lane = lax.iota(jnp.int32, 16)
    zi = jnp.zeros((16,), jnp.int32)
    zf = jnp.zeros((16,), jnp.float32)
    one = jnp.int32(1)
    zero = jnp.int32(0)

    for h in range(2):
        offv[h][pl.ds(0, 16)] = zi

    def emit(off, extra_mask):
        d = bdst[pl.ds(off, 16)]
        sr = bsrc[pl.ds(off, 16)]
        v = bval[pl.ds(off, 16)]
        for h in range(2):
            ld = d - h * NU
            m = (ld >= 0) & (ld < NU)
            if extra_mask is not None:
                m = m & extra_mask
            cnt = cntv[h][pl.ds(0, 16)][0]
            mi = jnp.where(m, one, zero)
            csum = plsc.cumsum(mi)
            idx = jnp.where(m, cnt + csum - mi, BLK + lane)
            plsc.store_scatter(ssb[h], [idx], sr)
            plsc.store_scatter(sab[h], [idx], ld)
            plsc.store_scatter(svb[h], [idx], v)
            cntv[h][pl.ds(0, 16)] = jnp.full((16,), cnt + csum[15],
                                             jnp.int32)

    def block(blk, _):
        ebase = pl.multiple_of(w * 25000 + blk * BLK, 8)
        pltpu.sync_copy(ei_hbm.at[0, pl.ds(ebase, BLK)],
                        bsrc.at[pl.ds(0, BLK)])
        pltpu.sync_copy(ei_hbm.at[1, pl.ds(ebase, BLK)],
                        bdst.at[pl.ds(0, BLK)])
        pltpu.sync_copy(ev_hbm.at[pl.ds(ebase, BLK)],
                        bval.at[pl.ds(0, BLK)])
        for h in range(2):
            cntv[h][pl.ds(0, 16)] = zi

        def grp(g, _):
            emit(g * 16, None)
            return 0

        lax.fori_loop(0, 61, grp, 0)
        emit(976, None)
        emit(984, lane >= 8)

        for h in range(2):
            cnt = cntv[h][pl.ds(0, 16)][0]
            tidx = cnt + lane
            plsc.store_scatter(ssb[h], [tidx], zi)
            plsc.store_scatter(sab[h], [tidx], zi)
            plsc.store_scatter(svb[h], [tidx], zf)
            cnt = (cnt + 7) & (-8)
            off = offv[h][pl.ds(0, 16)][0]
            sbase = pl.multiple_of((h * 32 + w) * CAP + off, 8)
            pltpu.sync_copy(ssb[h], psrc_hbm.at[pl.ds(sbase, STG)])
            pltpu.sync_copy(sab[h], padj_hbm.at[pl.ds(sbase, STG)])
            pltpu.sync_copy(svb[h], pval_hbm.at[pl.ds(sbase, STG)])
            offv[h][pl.ds(0, 16)] = jnp.full((16,), off + cnt, jnp.int32)
        return 0

    lax.fori_loop(0, 25, block, 0)

    def zgrp(g, _):
        ssb[0][pl.ds(g * 16, 16)] = zi
        sab[0][pl.ds(g * 16, 16)] = zi
        svb[0][pl.ds(g * 16, 16)] = zf
        return 0

    lax.fori_loop(0, 16, zgrp, 0)
    for h in range(2):
        off = offv[h][pl.ds(0, 16)][0]
        sbase = pl.multiple_of((h * 32 + w) * CAP + off, 8)
        pltpu.sync_copy(ssb[0].at[pl.ds(0, 256)],
                        psrc_hbm.at[pl.ds(sbase, 256)])
        pltpu.sync_copy(sab[0].at[pl.ds(0, 256)],
                        padj_hbm.at[pl.ds(sbase, 256)])
        pltpu.sync_copy(svb[0].at[pl.ds(0, 256)],
                        pval_hbm.at[pl.ds(sbase, 256)])
        ncnk = ((off + 255) & (-256)) >> 7
        cbuf[pl.ds(0, 16)] = jnp.full((16,), ncnk, jnp.int32)
        pltpu.sync_copy(cbuf, pcnt_hbm.at[pl.ds((h * 32 + w) * 16, 16)])


_prep_call = pl.kernel(
    _prep_body,
    out_type=(jax.ShapeDtypeStruct((2 * 32 * CAP,), jnp.int32),
              jax.ShapeDtypeStruct((2 * 32 * CAP,), jnp.int32),
              jax.ShapeDtypeStruct((2 * 32 * CAP,), jnp.float32),
              jax.ShapeDtypeStruct((1024,), jnp.int32)),
    mesh=_MESH,
    compiler_params=_SC_PARAMS,
    scratch_types=[
        pltpu.VMEM((BLK,), jnp.int32),
        pltpu.VMEM((BLK,), jnp.int32),
        pltpu.VMEM((BLK,), jnp.float32),
        pltpu.VMEM((STG,), jnp.int32),
        pltpu.VMEM((STG,), jnp.int32),
        pltpu.VMEM((STG,), jnp.float32),
        pltpu.VMEM((STG,), jnp.int32),
        pltpu.VMEM((STG,), jnp.int32),
        pltpu.VMEM((STG,), jnp.float32),
        pltpu.VMEM((16,), jnp.int32),
        pltpu.VMEM((16,), jnp.int32),
        pltpu.VMEM((16,), jnp.int32),
        pltpu.VMEM((16,), jnp.int32),
        pltpu.VMEM((16,), jnp.int32),
    ],
)

CH = 128
SLICE = 1568


def _layer_body(emb_hbm, psrc_hbm, padj_hbm, pval_hbm, pcnt_hbm, zeros_hbm,
                out_hbm, acc_sh, trash_sh, pbuf, src0, src1, val0, val1,
                adj0, adj1, sadj0, sadj1, rows0, rows1, isem0, isem1,
                gsem0, gsem1, ssem0, ssem1):
    c = lax.axis_index("c")
    s = lax.axis_index("s")
    lo = c * NU
    srcb = (src0, src1)
    valb = (val0, val1)
    adjb = (adj0, adj1)
    sadjb = (sadj0, sadj1)
    rowsb = (rows0, rows1)
    isem = (isem0, isem1)
    gsem = (gsem0, gsem1)
    ssem = (ssem0, ssem1)

    pltpu.sync_copy(pcnt_hbm, pbuf)
    ks = (c * 32 + 2 * s) * 16
    na = pbuf[pl.ds(ks, 16)][0]
    nb = pbuf[pl.ds(ks + 16, 16)][0]
    nt = na + nb
    abase = (c * 32 + 2 * s) * CAP
    bbase = abase + CAP

    def idx_start(jj, b):
        jc = jnp.minimum(jj, nt - 1)
        ebase = pl.multiple_of(
            jnp.where(jc < na, abase + jc * CH, bbase + (jc - na) * CH), 8)
        pltpu.async_copy(psrc_hbm.at[pl.ds(ebase, CH)], srcb[b], isem[b])
        pltpu.async_copy(padj_hbm.at[pl.ds(ebase, CH)], adjb[b], isem[b])
        pltpu.async_copy(pval_hbm.at[pl.ds(ebase, CH)], valb[b], isem[b])

    def idx_wait(b):
        pltpu.make_async_copy(psrc_hbm.at[pl.ds(0, CH)], srcb[b],
                              isem[b]).wait()
        pltpu.make_async_copy(padj_hbm.at[pl.ds(0, CH)], adjb[b],
                              isem[b]).wait()
        pltpu.make_async_copy(pval_hbm.at[pl.ds(0, CH)], valb[b],
                              isem[b]).wait()

    def gather_start(b):
        pltpu.async_copy(emb_hbm.at[srcb[b]], rowsb[b], gsem[b])

    def gather_wait(b):
        pltpu.make_async_copy(emb_hbm.at[pl.ds(0, CH)], rowsb[b],
                              gsem[b]).wait()

    def compute(b):
        def sgrp(g, _):
            vv = valb[b][pl.ds(g * 16, 16)]
            sadjb[b][pl.ds(g * 16, 16)] = adjb[b][pl.ds(g * 16, 16)]
            for l in range(16):
                v = vv[l]
                e = g * 16 + l
                for q in range(4):
                    rowsb[b][e, pl.ds(q * 16, 16)] = (
                        rowsb[b][e, pl.ds(q * 16, 16)] * v)
            return 0

        lax.fori_loop(0, CH // 16, sgrp, 0)

    def scatter_wait(b):
        pltpu.make_async_copy(rowsb[b], acc_sh.at[pl.ds(0, CH)],
                              ssem[b]).wait()

    def section(j, b):
        idx_wait(1 - b)
        scatter_wait(1 - b)
        gather_start(1 - b)
        gather_wait(b)
        compute(b)
        idx_start(j + 2, b)
        pltpu.async_copy(rowsb[b], acc_sh.at[sadjb[b]], ssem[b], add=True)

    zb = pl.multiple_of(jnp.minimum(s * SLICE, NU - SLICE), 8)
    idx_start(0, 0)
    idx_start(1, 1)
    zi = jnp.zeros((16,), jnp.int32)

    def zadj(g, _):
        sadjb[0][pl.ds(g * 16, 16)] = zi
        sadjb[1][pl.ds(g * 16, 16)] = zi
        return 0

    lax.fori_loop(0, CH // 16, zadj, 0)
    pltpu.async_copy(rowsb[0], trash_sh.at[sadjb[0]], ssem[0], add=True)
    pltpu.async_copy(rowsb[1], trash_sh.at[sadjb[1]], ssem[1], add=True)
    pltpu.sync_copy(zeros_hbm, acc_sh.at[pl.ds(zb, SLICE)])
    idx_wait(0)
    gather_start(0)
    plsc.subcore_barrier()

    def pair(g, _):
        section(2 * g, 0)
        section(2 * g + 1, 1)
        return 0

    lax.fori_loop(0, nt // 2, pair, 0)
    gather_wait(0)
    idx_wait(1)
    scatter_wait(0)
    scatter_wait(1)
    plsc.subcore_barrier()
    wb = pl.multiple_of(jnp.minimum(s * SLICE, NU - SLICE), 8)
    pltpu.sync_copy(acc_sh.at[pl.ds(wb, SLICE)],
                    out_hbm.at[pl.ds(lo + wb, SLICE)])


_layer_call = pl.kernel(
    _layer_body,
    out_type=jax.ShapeDtypeStruct((NN, D), jnp.float32),
    mesh=_MESH,
    compiler_params=_SC_PARAMS,
    scratch_types=[
        pltpu.VMEM_SHARED((NU, D), jnp.float32),
        pltpu.VMEM_SHARED((16, D), jnp.float32),
        pltpu.VMEM((1024,), jnp.int32),
        pltpu.VMEM((CH,), jnp.int32),
        pltpu.VMEM((CH,), jnp.int32),
        pltpu.VMEM((CH,), jnp.float32),
        pltpu.VMEM((CH,), jnp.float32),
        pltpu.VMEM((CH,), jnp.int32),
        pltpu.VMEM((CH,), jnp.int32),
        pltpu.VMEM((CH,), jnp.int32),
        pltpu.VMEM((CH,), jnp.int32),
        pltpu.VMEM((CH, D), jnp.float32),
        pltpu.VMEM((CH, D), jnp.float32),
        pltpu.SemaphoreType.DMA,
        pltpu.SemaphoreType.DMA,
        pltpu.SemaphoreType.DMA,
        pltpu.SemaphoreType.DMA,
        pltpu.SemaphoreType.DMA,
        pltpu.SemaphoreType.DMA,
    ],
)

MR = 200


def _mean_body(e0_hbm, e1_hbm, e2_hbm, e3_hbm, out_hbm,
               b0, b1, b2, b3, ob):
    c = lax.axis_index("c")
    s = lax.axis_index("s")
    half = c * NU
    for t in range(8):
        base = pl.multiple_of(jnp.minimum((s * 8 + t) * MR, NU - MR), 8)
        pltpu.sync_copy(e0_hbm.at[pl.ds(half + base, MR)], b0)
        pltpu.sync_copy(e1_hbm.at[pl.ds(half + base, MR)], b1)
        pltpu.sync_copy(e2_hbm.at[pl.ds(half + base, MR)], b2)
        pltpu.sync_copy(e3_hbm.at[pl.ds(half + base, MR)], b3)

        def mrow(i, _):
            for q in range(4):
                dq = pl.ds(q * 16, 16)
                ob[i, dq] = (b0[i, dq] + b1[i, dq] + b2[i, dq]
                             + b3[i, dq]) * jnp.float32(0.25)
            return 0

        lax.fori_loop(0, MR, mrow, 0)

        pltpu.sync_copy(ob, out_hbm.at[pl.ds(half + base, MR)])


_mean_call = pl.kernel(
    _mean_body,
    out_type=jax.ShapeDtypeStruct((NN, D), jnp.float32),
    mesh=_MESH,
    compiler_params=_SC_PARAMS,
    scratch_types=[pltpu.VMEM((MR, D), jnp.float32) for _ in range(5)],
)


def kernel(embedding_user, embedding_item, embedding_entity, item_entities,
           edge_index, edge_values):
    ief = item_entities.reshape(-1).astype(jnp.int32)
    ei = edge_index.astype(jnp.int32)
    zeros = jnp.zeros((SLICE, D), jnp.float32)
    psrc, padj, pval, pcnt = _prep_call(ei, edge_values)
    e0 = _kg_call(embedding_user, embedding_item, embedding_entity, ief)
    e1 = _layer_call(e0, psrc, padj, pval, pcnt, zeros)
    e2 = _layer_call(e1, psrc, padj, pval, pcnt, zeros)
    e3 = _layer_call(e2, psrc, padj, pval, pcnt, zeros)
    light = _mean_call(e0, e1, e2, e3)
    return light[:NU], light[NU:]


# speedup vs baseline: 1.6004x; 1.6004x over previous
import functools

import jax
import jax.numpy as jnp
from jax import lax
from jax.experimental import pallas as pl
from jax.experimental.pallas import tpu as pltpu
from jax.experimental.pallas import tpu_sc as plsc

NU = 25000
NI = 25000
NN = NU + NI
D = 64
K = 8
PAD = 100000
NL = 3
E = 800000
NC = 2
NS = 16

_MESH = plsc.VectorSubcoreMesh(core_axis_name="c", subcore_axis_name="s")

IG = 56
NG = 14
IW = IG * NG


def _kg_body(user_hbm, item_hbm, ent_hbm, ief_hbm, e0_hbm,
             idx_v, rows_v, item_v, out_v, ubuf_v):
    c = lax.axis_index("c")
    s = lax.axis_index("s")
    w = s * NC + c

    ub = pl.multiple_of(jnp.minimum(w * IW, NU - IW), 8)
    for t in range(2):
        pltpu.sync_copy(user_hbm.at[pl.ds(ub + t * 392, 392)], ubuf_v)
        pltpu.sync_copy(ubuf_v, e0_hbm.at[pl.ds(ub + t * 392, 392)])

    base = pl.multiple_of(jnp.minimum(w * IW, NI - IW), 8)

    def group(g, _):
        ib = pl.multiple_of(base + g * IG, 8)
        pltpu.sync_copy(ief_hbm.at[pl.ds(ib * K, IG * K)],
                        idx_v.at[pl.ds(0, IG * K)])
        pltpu.sync_copy(ent_hbm.at[idx_v.at[pl.ds(0, IG * K)]], rows_v)
        pltpu.sync_copy(item_hbm.at[pl.ds(ib, IG)], item_v)

        def item(i, _):
            acc = [jnp.zeros((16,), jnp.float32) for _ in range(4)]
            den = jnp.float32(0.0)
            nv = idx_v[pl.ds(i * K, 16)]
            mv = jnp.where(nv != PAD, jnp.float32(1.0), jnp.float32(0.0))
            for k in range(K):
                m = mv[k]
                den = den + m
                for q in range(4):
                    acc[q] = acc[q] + m * rows_v[i * K + k, pl.ds(q * 16, 16)]
            denv = jnp.full((16,), den, jnp.float32)
            invv = jnp.where(denv > 0.0, jnp.float32(1.0) / denv,
                             jnp.float32(0.0))
            for q in range(4):
                out_v[i, pl.ds(q * 16, 16)] = (
                    acc[q] * invv + item_v[i, pl.ds(q * 16, 16)])
            return 0

        lax.fori_loop(0, IG, item, 0)
        pltpu.sync_copy(out_v, e0_hbm.at[pl.ds(NU + ib, IG)])
        return 0

    lax.fori_loop(0, NG, group, 0)


_SC_PARAMS = pltpu.CompilerParams(use_tc_tiling_on_sc=False)

_kg_call = pl.kernel(
    _kg_body,
    out_type=jax.ShapeDtypeStruct((NN, D), jnp.float32),
    mesh=_MESH,
    compiler_params=_SC_PARAMS,
    scratch_types=[
        pltpu.VMEM((IG * K + 8,), jnp.int32),
        pltpu.VMEM((IG * K, D), jnp.float32),
        pltpu.VMEM((IG, D), jnp.float32),
        pltpu.VMEM((IG, D), jnp.float32),
        pltpu.VMEM((392, D), jnp.float32),
    ],
)

CH = 200
PER = E // NS
NCHUNK = PER // CH
SLICE = 1568


def _layer_body(emb_hbm, ei_hbm, ev_hbm, zeros_hbm, out_hbm,
                acc_sh, src0, src1, dst0, dst1, val0, val1, adj0, adj1,
                rows0, rows1, isem0, isem1, gsem0, gsem1):
    c = lax.axis_index("c")
    s = lax.axis_index("s")
    lo = c * NU
    sink = s * 1563
    srcb = (src0, src1)
    dstb = (dst0, dst1)
    valb = (val0, val1)
    adjb = (adj0, adj1)
    rowsb = (rows0, rows1)
    isem = (isem0, isem1)
    gsem = (gsem0, gsem1)

    def idx_start(jj, b):
        jc = jnp.minimum(jj, NCHUNK - 1)
        ebase = pl.multiple_of(s * PER + jc * CH, 8)
        pltpu.async_copy(ei_hbm.at[0, pl.ds(ebase, CH)], srcb[b], isem[b])
        pltpu.async_copy(ei_hbm.at[1, pl.ds(ebase, CH)], dstb[b], isem[b])
        pltpu.async_copy(ev_hbm.at[pl.ds(ebase, CH)], valb[b], isem[b])

    def idx_wait(b):
        pltpu.make_async_copy(ei_hbm.at[0, pl.ds(0, CH)], srcb[b],
                              isem[b]).wait()
        pltpu.make_async_copy(ei_hbm.at[1, pl.ds(0, CH)], dstb[b],
                              isem[b]).wait()
        pltpu.make_async_copy(ev_hbm.at[pl.ds(0, CH)], valb[b],
                              isem[b]).wait()

    def gather_start(b):
        pltpu.async_copy(emb_hbm.at[srcb[b]], rowsb[b], gsem[b])

    def gather_wait(b):
        pltpu.make_async_copy(emb_hbm.at[pl.ds(0, CH)], rowsb[b],
                              gsem[b]).wait()

    def compute(b):
        def adj_group(off, lanes):
            d = dstb[b][pl.ds(off, 16)]
            ld = d - lo
            inb = (ld >= 0) & (ld < NU)
            adjb[b][pl.ds(off, 16)] = jnp.where(inb, ld, sink)
            mf = jnp.where(inb, jnp.float32(1.0), jnp.float32(0.0))
            valb[b][pl.ds(off, 16)] = valb[b][pl.ds(off, 16)] * mf

        def scale_group(off, lanes):
            vv = valb[b][pl.ds(off, 16)]
            for l in lanes:
                v = vv[l]
                e = off + l
                for q in range(4):
                    rowsb[b][e, pl.ds(q * 16, 16)] = (
                        rowsb[b][e, pl.ds(q * 16, 16)] * v)

        def grp(g, _):
            adj_group(g * 16, range(16))
            return 0

        lax.fori_loop(0, CH // 16, grp, 0)
        adj_group(CH - 16, range(16))

        def sgrp(g, _):
            scale_group(g * 16, range(16))
            return 0

        lax.fori_loop(0, CH // 16, sgrp, 0)
        scale_group(CH - 16, range(16 - (CH - CH // 16 * 16), 16))

    def section(j, b):
        idx_wait(1 - b)
        gather_start(1 - b)
        gather_wait(b)
        compute(b)
        idx_start(j + 2, b)
        pltpu.sync_copy(rowsb[b], acc_sh.at[adjb[b]], add=True)

    zb = pl.multiple_of(jnp.minimum(s * SLICE, NU - SLICE), 8)
    idx_start(0, 0)
    idx_start(1, 1)
    pltpu.sync_copy(zeros_hbm, acc_sh.at[pl.ds(zb, SLICE)])
    idx_wait(0)
    gather_start(0)
    plsc.subcore_barrier()

    def pair(g, _):
        section(2 * g, 0)
        section(2 * g + 1, 1)
        return 0

    lax.fori_loop(0, NCHUNK // 2, pair, 0)
    gather_wait(0)
    idx_wait(1)
    plsc.subcore_barrier()
    wb = pl.multiple_of(jnp.minimum(s * SLICE, NU - SLICE), 8)
    pltpu.sync_copy(acc_sh.at[pl.ds(wb, SLICE)],
                    out_hbm.at[pl.ds(lo + wb, SLICE)])


_layer_call = pl.kernel(
    _layer_body,
    out_type=jax.ShapeDtypeStruct((NN, D), jnp.float32),
    mesh=_MESH,
    compiler_params=_SC_PARAMS,
    scratch_types=[
        pltpu.VMEM_SHARED((NU, D), jnp.float32),
        pltpu.VMEM((CH,), jnp.int32),
        pltpu.VMEM((CH,), jnp.int32),
        pltpu.VMEM((CH,), jnp.int32),
        pltpu.VMEM((CH,), jnp.int32),
        pltpu.VMEM((CH,), jnp.float32),
        pltpu.VMEM((CH,), jnp.float32),
        pltpu.VMEM((CH,), jnp.int32),
        pltpu.VMEM((CH,), jnp.int32),
        pltpu.VMEM((CH, D), jnp.float32),
        pltpu.VMEM((CH, D), jnp.float32),
        pltpu.SemaphoreType.DMA,
        pltpu.SemaphoreType.DMA,
        pltpu.SemaphoreType.DMA,
        pltpu.SemaphoreType.DMA,
    ],
)

MR = 200


def _mean_body(e0_hbm, e1_hbm, e2_hbm, e3_hbm, out_hbm,
               b0, b1, b2, b3, ob):
    c = lax.axis_index("c")
    s = lax.axis_index("s")
    half = c * NU
    for t in range(8):
        base = pl.multiple_of(jnp.minimum((s * 8 + t) * MR, NU - MR), 8)
        pltpu.sync_copy(e0_hbm.at[pl.ds(half + base, MR)], b0)
        pltpu.sync_copy(e1_hbm.at[pl.ds(half + base, MR)], b1)
        pltpu.sync_copy(e2_hbm.at[pl.ds(half + base, MR)], b2)
        pltpu.sync_copy(e3_hbm.at[pl.ds(half + base, MR)], b3)

        def mrow(i, _):
            for q in range(4):
                dq = pl.ds(q * 16, 16)
                ob[i, dq] = (b0[i, dq] + b1[i, dq] + b2[i, dq]
                             + b3[i, dq]) * jnp.float32(0.25)
            return 0

        lax.fori_loop(0, MR, mrow, 0)

        pltpu.sync_copy(ob, out_hbm.at[pl.ds(half + base, MR)])


_mean_call = pl.kernel(
    _mean_body,
    out_type=jax.ShapeDtypeStruct((NN, D), jnp.float32),
    mesh=_MESH,
    compiler_params=_SC_PARAMS,
    scratch_types=[pltpu.VMEM((MR, D), jnp.float32) for _ in range(5)],
)


def kernel(embedding_user, embedding_item, embedding_entity, item_entities,
           edge_index, edge_values):
    ief = item_entities.reshape(-1).astype(jnp.int32)
    ei = edge_index.astype(jnp.int32)
    zeros = jnp.zeros((SLICE, D), jnp.float32)
    e0 = _kg_call(embedding_user, embedding_item, embedding_entity, ief)
    e1 = _layer_call(e0, ei, edge_values, zeros)
    e2 = _layer_call(e1, ei, edge_values, zeros)
    e3 = _layer_call(e2, ei, edge_values, zeros)
    light = _mean_call(e0, e1, e2, e3)
    return light[:NU], light[NU:]
